# baseline (speedup 1.0000x reference)
import functools

import jax
import jax.numpy as jnp
from jax import lax
from jax.experimental import pallas as pl
from jax.experimental.pallas import tpu as pltpu
from jax.experimental.pallas import tpu_sc as plsc

K_TOP = 8
LANES = 16
NUM_CORES = 2
NUM_SUBCORES = 16
NUM_WORKERS = NUM_CORES * NUM_SUBCORES
CHUNK = 256
BLOCK = 16
CPW = 128


def _cas(a, b):
    return jnp.maximum(a, b), jnp.minimum(a, b)


def _merge22(A, B):
    c0, x = _cas(A[0], B[0])
    y, c3 = _cas(A[1], B[1])
    c1, c2 = _cas(x, y)
    return [c0, c1, c2, c3]


def _merge44(A, B):
    E = _merge22([A[0], A[2]], [B[0], B[2]])
    O = _merge22([A[1], A[3]], [B[1], B[3]])
    c1, c2 = _cas(O[0], E[1])
    c3, c4 = _cas(O[1], E[2])
    c5, c6 = _cas(O[2], E[3])
    return [E[0], c1, c2, c3, c4, c5, c6, O[3]]


def _bitonic_clean8(x):
    y = [None] * 8
    for i in range(4):
        y[i], y[i + 4] = _cas(x[i], x[i + 4])
    z = [None] * 8
    for h in (0, 4):
        for i in range(2):
            z[h + i], z[h + i + 2] = _cas(y[h + i], y[h + i + 2])
    w = [None] * 8
    for h in (0, 2, 4, 6):
        w[h], w[h + 1] = _cas(z[h], z[h + 1])
    return w


def _merge_top8(A, B):
    return _bitonic_clean8([jnp.maximum(A[i], B[7 - i]) for i in range(8)])


def _block_top8(v):
    S2 = [_cas(v[2 * j], v[2 * j + 1]) for j in range(8)]
    S4 = [_merge22(S2[2 * j], S2[2 * j + 1]) for j in range(4)]
    S8a = _merge44(S4[0], S4[1])
    S8b = _merge44(S4[2], S4[3])
    return _merge_top8(S8a, S8b)


def _topk_sc(x3):
    B, S, C = x3.shape
    CG = C // LANES
    GPW = CPW // LANES
    WPB = NUM_WORKERS // B
    NCHUNK = S // CHUNK

    x = x3.reshape(B * S, C)
    mesh = plsc.VectorSubcoreMesh(core_axis_name="c", subcore_axis_name="s")

    @functools.partial(
        pl.kernel,
        out_type=jax.ShapeDtypeStruct((B, CG, K_TOP * LANES), jnp.float32),
        mesh=mesh,
        scratch_types=[
            pltpu.VMEM((CHUNK, CPW), jnp.float32),
            pltpu.VMEM((CHUNK, CPW), jnp.float32),
            pltpu.VMEM((GPW, K_TOP * LANES), jnp.float32),
            pltpu.SemaphoreType.DMA,
            pltpu.SemaphoreType.DMA,
        ],
        compiler_params=pltpu.CompilerParams(use_tc_tiling_on_sc=True),
    )
    def k(x_hbm, out_hbm, buf0, buf1, acc, sem0, sem1):
        wid = lax.axis_index("s") * NUM_CORES + lax.axis_index("c")
        b = wid // WPB
        seg = wid % WPB
        g0 = seg * GPW
        c0 = seg * CPW
        row0 = b * S

        neg = jnp.full((LANES,), -jnp.inf, dtype=jnp.float32)
        for g in range(GPW):
            for kk in range(K_TOP):
                acc[g, pl.ds(kk * LANES, LANES)] = neg

        pltpu.async_copy(
            x_hbm.at[pl.ds(row0, CHUNK), pl.ds(c0, CPW)], buf0, sem0)
        pltpu.async_copy(
            x_hbm.at[pl.ds(row0 + CHUNK, CHUNK), pl.ds(c0, CPW)], buf1, sem1)

        def process(buf):
            for g in range(GPW):
                def blk_body(i, t, g=g, buf=buf):
                    va = [buf[i * 2 * BLOCK + u, pl.ds(g * LANES, LANES)]
                          for u in range(BLOCK)]
                    vb = [buf[i * 2 * BLOCK + BLOCK + u,
                              pl.ds(g * LANES, LANES)]
                          for u in range(BLOCK)]
                    sa = _block_top8(va)
                    sb = _block_top8(vb)
                    u8 = _merge_top8(sa, sb)
                    return tuple(_merge_top8(list(t), u8))

                t = tuple(acc[g, pl.ds(kk * LANES, LANES)]
                          for kk in range(K_TOP))
                t = lax.fori_loop(0, CHUNK // (2 * BLOCK), blk_body, t)
                for kk in range(K_TOP):
                    acc[g, pl.ds(kk * LANES, LANES)] = t[kk]

        @pl.loop(0, NCHUNK, step=2)
        def _(ci):
            for j, (buf, sem) in enumerate(((buf0, sem0), (buf1, sem1))):
                cc = ci + j
                pltpu.make_async_copy(
                    x_hbm.at[pl.ds(row0, CHUNK), pl.ds(c0, CPW)], buf, sem
                ).wait()
                process(buf)

                @pl.when(cc + 2 < NCHUNK)
                def _(buf=buf, sem=sem, cc=cc):
                    pltpu.async_copy(
                        x_hbm.at[pl.ds(row0 + (cc + 2) * CHUNK, CHUNK),
                                 pl.ds(c0, CPW)],
                        buf, sem)

        pltpu.sync_copy(acc, out_hbm.at[b, pl.ds(g0, GPW)])

    return k(x)


def kernel(inputs):
    B, S, C = inputs.shape
    out3 = _topk_sc(inputs)
    out4 = out3.reshape(B, C // LANES, K_TOP, LANES)
    return jnp.transpose(out4, (0, 1, 3, 2)).reshape(B, C * K_TOP)


# speedup vs baseline: 1.2021x; 1.2021x over previous
import functools

import jax
import jax.numpy as jnp
from jax import lax
from jax.experimental import pallas as pl
from jax.experimental.pallas import tpu as pltpu
from jax.experimental.pallas import tpu_sc as plsc

K_TOP = 8
LANES = 16
NUM_CORES = 2
NUM_SUBCORES = 16
NUM_WORKERS = NUM_CORES * NUM_SUBCORES
CHUNK = 256
BLOCK = 16
CPW = 128


def _cas(a, b):
    return jnp.maximum(a, b), jnp.minimum(a, b)


def _merge22(A, B):
    c0, x = _cas(A[0], B[0])
    y, c3 = _cas(A[1], B[1])
    c1, c2 = _cas(x, y)
    return [c0, c1, c2, c3]


def _merge44(A, B):
    E = _merge22([A[0], A[2]], [B[0], B[2]])
    O = _merge22([A[1], A[3]], [B[1], B[3]])
    c1, c2 = _cas(O[0], E[1])
    c3, c4 = _cas(O[1], E[2])
    c5, c6 = _cas(O[2], E[3])
    return [E[0], c1, c2, c3, c4, c5, c6, O[3]]


def _bitonic_clean8(x):
    y = [None] * 8
    for i in range(4):
        y[i], y[i + 4] = _cas(x[i], x[i + 4])
    z = [None] * 8
    for h in (0, 4):
        for i in range(2):
            z[h + i], z[h + i + 2] = _cas(y[h + i], y[h + i + 2])
    w = [None] * 8
    for h in (0, 2, 4, 6):
        w[h], w[h + 1] = _cas(z[h], z[h + 1])
    return w


def _merge_top8(A, B):
    return _bitonic_clean8([jnp.maximum(A[i], B[7 - i]) for i in range(8)])


def _block_top8(v):
    S2 = [_cas(v[2 * j], v[2 * j + 1]) for j in range(8)]
    S4 = [_merge22(S2[2 * j], S2[2 * j + 1]) for j in range(4)]
    S8a = _merge44(S4[0], S4[1])
    S8b = _merge44(S4[2], S4[3])
    return _merge_top8(S8a, S8b)


def _topk_sc(x3):
    B, S, C = x3.shape
    CG = C // LANES
    GPW = CPW // LANES
    WPB = NUM_WORKERS // B
    NCHUNK = S // CHUNK

    x = x3.reshape(B * S, C)
    mesh = plsc.VectorSubcoreMesh(core_axis_name="c", subcore_axis_name="s")

    @functools.partial(
        pl.kernel,
        out_type=jax.ShapeDtypeStruct((B, C * K_TOP), jnp.float32),
        mesh=mesh,
        scratch_types=[
            pltpu.VMEM((CHUNK, CPW), jnp.float32),
            pltpu.VMEM((CHUNK, CPW), jnp.float32),
            pltpu.VMEM((GPW, K_TOP * LANES), jnp.float32),
            pltpu.VMEM((GPW * K_TOP * LANES,), jnp.float32),
            pltpu.SemaphoreType.DMA,
            pltpu.SemaphoreType.DMA,
        ],
        compiler_params=pltpu.CompilerParams(
            use_tc_tiling_on_sc=True, needs_layout_passes=False),
    )
    def k(x_hbm, out_hbm, buf0, buf1, acc, accp, sem0, sem1):
        wid = lax.axis_index("s") * NUM_CORES + lax.axis_index("c")
        b = wid // WPB
        seg = wid % WPB
        g0 = seg * GPW
        c0 = seg * CPW
        row0 = b * S

        neg = jnp.full((LANES,), -jnp.inf, dtype=jnp.float32)
        for g in range(GPW):
            for kk in range(K_TOP):
                acc[g, pl.ds(kk * LANES, LANES)] = neg

        pltpu.async_copy(
            x_hbm.at[pl.ds(row0, CHUNK), pl.ds(c0, CPW)], buf0, sem0)
        pltpu.async_copy(
            x_hbm.at[pl.ds(row0 + CHUNK, CHUNK), pl.ds(c0, CPW)], buf1, sem1)

        def process(buf):
            for g in range(GPW):
                def blk_body(i, t, g=g, buf=buf):
                    v = [buf[i * BLOCK + u, pl.ds(g * LANES, LANES)]
                         for u in range(BLOCK)]
                    s = _block_top8(v)
                    return tuple(_merge_top8(list(t), s))

                t = tuple(acc[g, pl.ds(kk * LANES, LANES)]
                          for kk in range(K_TOP))
                t = lax.fori_loop(0, CHUNK // BLOCK, blk_body, t)
                for kk in range(K_TOP):
                    acc[g, pl.ds(kk * LANES, LANES)] = t[kk]

        @pl.loop(0, NCHUNK, step=2)
        def _(ci):
            for j, (buf, sem) in enumerate(((buf0, sem0), (buf1, sem1))):
                cc = ci + j
                pltpu.make_async_copy(
                    x_hbm.at[pl.ds(row0, CHUNK), pl.ds(c0, CPW)], buf, sem
                ).wait()
                process(buf)

                @pl.when(cc + 2 < NCHUNK)
                def _(buf=buf, sem=sem, cc=cc):
                    pltpu.async_copy(
                        x_hbm.at[pl.ds(row0 + (cc + 2) * CHUNK, CHUNK),
                                 pl.ds(c0, CPW)],
                        buf, sem)

        iota = lax.iota(jnp.int32, LANES)
        for g in range(GPW):
            for kk in range(K_TOP):
                plsc.store_scatter(
                    accp,
                    [iota * K_TOP + (g * LANES * K_TOP + kk)],
                    acc[g, pl.ds(kk * LANES, LANES)])
        pltpu.sync_copy(
            accp, out_hbm.at[b, pl.ds(c0 * K_TOP, CPW * K_TOP)])

    return k(x)


def kernel(inputs):
    return _topk_sc(inputs)


# speedup vs baseline: 1.5942x; 1.3262x over previous
import functools

import jax
import jax.numpy as jnp
from jax import lax
from jax.experimental import pallas as pl
from jax.experimental.pallas import tpu as pltpu
from jax.experimental.pallas import tpu_sc as plsc

K_TOP = 8
LANES = 16
NUM_CORES = 2
NUM_SUBCORES = 16
NUM_WORKERS = NUM_CORES * NUM_SUBCORES
BLOCK = 16
C_SC = 256
CHUNK = 256


def _cas(a, b):
    return jnp.maximum(a, b), jnp.minimum(a, b)


def _merge22(A, B):
    c0, x = _cas(A[0], B[0])
    y, c3 = _cas(A[1], B[1])
    c1, c2 = _cas(x, y)
    return [c0, c1, c2, c3]


def _merge44(A, B):
    E = _merge22([A[0], A[2]], [B[0], B[2]])
    O = _merge22([A[1], A[3]], [B[1], B[3]])
    c1, c2 = _cas(O[0], E[1])
    c3, c4 = _cas(O[1], E[2])
    c5, c6 = _cas(O[2], E[3])
    return [E[0], c1, c2, c3, c4, c5, c6, O[3]]


def _bitonic_clean8(x):
    y = [None] * 8
    for i in range(4):
        y[i], y[i + 4] = _cas(x[i], x[i + 4])
    z = [None] * 8
    for h in (0, 4):
        for i in range(2):
            z[h + i], z[h + i + 2] = _cas(y[h + i], y[h + i + 2])
    w = [None] * 8
    for h in (0, 2, 4, 6):
        w[h], w[h + 1] = _cas(z[h], z[h + 1])
    return w


def _merge_top8(A, B):
    return _bitonic_clean8([jnp.maximum(A[i], B[7 - i]) for i in range(8)])


def _block_top8(v):
    S2 = [_cas(v[2 * j], v[2 * j + 1]) for j in range(8)]
    S4 = [_merge22(S2[2 * j], S2[2 * j + 1]) for j in range(4)]
    S8a = _merge44(S4[0], S4[1])
    S8b = _merge44(S4[2], S4[3])
    return _merge_top8(S8a, S8b)


def _topk_sc(x3):
    B, S, C = x3.shape
    SLAB = 128
    QUARTERS = 4
    GPW = SLAB // LANES // QUARTERS
    SPB = C_SC // SLAB
    WPB = SPB * QUARTERS
    NCHUNK = S // CHUNK

    x = x3.reshape(B * S, C)
    mesh = plsc.VectorSubcoreMesh(core_axis_name="c", subcore_axis_name="s")

    @functools.partial(
        pl.kernel,
        out_type=jax.ShapeDtypeStruct((B, C_SC * K_TOP), jnp.float32),
        mesh=mesh,
        scratch_types=[
            pltpu.VMEM((CHUNK, SLAB), jnp.float32),
            pltpu.VMEM((CHUNK, SLAB), jnp.float32),
            pltpu.VMEM((GPW, K_TOP * LANES), jnp.float32),
            pltpu.VMEM((GPW * K_TOP * LANES,), jnp.float32),
            pltpu.SemaphoreType.DMA,
            pltpu.SemaphoreType.DMA,
        ],
        compiler_params=pltpu.CompilerParams(
            use_tc_tiling_on_sc=True, needs_layout_passes=False),
    )
    def k(x_hbm, out_hbm, buf0, buf1, acc, accp, sem0, sem1):
        wid = lax.axis_index("s") * NUM_CORES + lax.axis_index("c")
        b = wid // WPB
        slab = (wid % WPB) // QUARTERS
        q = wid % QUARTERS
        c0 = slab * SLAB
        row0 = b * S

        neg = jnp.full((LANES,), -jnp.inf, dtype=jnp.float32)
        for g in range(GPW):
            for kk in range(K_TOP):
                acc[g, pl.ds(kk * LANES, LANES)] = neg

        pltpu.async_copy(
            x_hbm.at[pl.ds(row0, CHUNK), pl.ds(c0, SLAB)], buf0, sem0)
        pltpu.async_copy(
            x_hbm.at[pl.ds(row0 + CHUNK, CHUNK), pl.ds(c0, SLAB)], buf1, sem1)

        def process(buf):
            for gp in range(SLAB // LANES):
                @pl.when(gp // GPW == q)
                def _(gp=gp, buf=buf):
                    g = gp % GPW

                    def blk_body(i, t, gp=gp, buf=buf):
                        v = [buf[i * BLOCK + u, pl.ds(gp * LANES, LANES)]
                             for u in range(BLOCK)]
                        s = _block_top8(v)
                        return tuple(_merge_top8(list(t), s))

                    t = tuple(acc[g, pl.ds(kk * LANES, LANES)]
                              for kk in range(K_TOP))
                    t = lax.fori_loop(0, CHUNK // BLOCK, blk_body, t)
                    for kk in range(K_TOP):
                        acc[g, pl.ds(kk * LANES, LANES)] = t[kk]

        @pl.loop(0, NCHUNK, step=2)
        def _(ci):
            for j, (buf, sem) in enumerate(((buf0, sem0), (buf1, sem1))):
                cc = ci + j
                pltpu.make_async_copy(
                    x_hbm.at[pl.ds(row0, CHUNK), pl.ds(c0, SLAB)], buf, sem
                ).wait()
                process(buf)

                @pl.when(cc + 2 < NCHUNK)
                def _(buf=buf, sem=sem, cc=cc):
                    pltpu.async_copy(
                        x_hbm.at[pl.ds(row0 + (cc + 2) * CHUNK, CHUNK),
                                 pl.ds(c0, SLAB)],
                        buf, sem)

        iota = lax.iota(jnp.int32, LANES)
        for g in range(GPW):
            for kk in range(K_TOP):
                plsc.store_scatter(
                    accp,
                    [iota * K_TOP + (g * LANES * K_TOP + kk)],
                    acc[g, pl.ds(kk * LANES, LANES)])
        pltpu.sync_copy(
            accp,
            out_hbm.at[b, pl.ds((c0 + q * LANES * GPW) * K_TOP,
                                GPW * LANES * K_TOP)])

    return k(x)


def _topk_tc(x3):
    B, S, C = x3.shape
    NCB = (C - C_SC) // 128

    def body(x_ref, o_ref):
        def blk(i, t):
            v = [x_ref[0, pl.ds(i * 8 * BLOCK + 8 * u, 8), :]
                 for u in range(BLOCK)]
            s = _block_top8(v)
            return tuple(_merge_top8(list(t), s))

        t = tuple(jnp.full((8, 128), -jnp.inf, dtype=jnp.float32)
                  for _ in range(K_TOP))
        t = list(lax.fori_loop(0, S // (8 * BLOCK), blk, t))
        for d in (1, 2, 4):
            tsh = [pltpu.roll(u, 8 - d, 0) for u in t]
            t = _merge_top8(t, tsh)
        for kk in range(K_TOP):
            o_ref[0, 0, pl.ds(kk, 1), :] = t[kk][0:1, :]

    out = pl.pallas_call(
        body,
        grid=(B, NCB),
        in_specs=[pl.BlockSpec((1, S, 128),
                               lambda i, j: (i, 0, j + C_SC // 128))],
        out_specs=pl.BlockSpec((1, 1, K_TOP, 128),
                               lambda i, j: (i, j, 0, 0)),
        out_shape=jax.ShapeDtypeStruct((B, NCB, K_TOP, 128), jnp.float32),
    )(x3)
    return jnp.transpose(out, (0, 1, 3, 2)).reshape(B, (C - C_SC) * K_TOP)


def kernel(inputs):
    sc_out = _topk_sc(inputs)
    tc_out = _topk_tc(inputs)
    return jnp.concatenate([sc_out, tc_out], axis=1)


# speedup vs baseline: 1.9517x; 1.2242x over previous
import functools

import jax
import jax.numpy as jnp
from jax import lax
from jax.experimental import pallas as pl
from jax.experimental.pallas import tpu as pltpu
from jax.experimental.pallas import tpu_sc as plsc

K_TOP = 8
LANES = 16
NUM_CORES = 2
NUM_SUBCORES = 16
NUM_WORKERS = NUM_CORES * NUM_SUBCORES
BLOCK = 16
C_SC = 256
CHUNK = 256


def _cas(a, b):
    return jnp.maximum(a, b), jnp.minimum(a, b)


def _merge22(A, B):
    c0, x = _cas(A[0], B[0])
    y, c3 = _cas(A[1], B[1])
    c1, c2 = _cas(x, y)
    return [c0, c1, c2, c3]


def _merge44(A, B):
    E = _merge22([A[0], A[2]], [B[0], B[2]])
    O = _merge22([A[1], A[3]], [B[1], B[3]])
    c1, c2 = _cas(O[0], E[1])
    c3, c4 = _cas(O[1], E[2])
    c5, c6 = _cas(O[2], E[3])
    return [E[0], c1, c2, c3, c4, c5, c6, O[3]]


def _bitonic_clean8(x):
    y = [None] * 8
    for i in range(4):
        y[i], y[i + 4] = _cas(x[i], x[i + 4])
    z = [None] * 8
    for h in (0, 4):
        for i in range(2):
            z[h + i], z[h + i + 2] = _cas(y[h + i], y[h + i + 2])
    w = [None] * 8
    for h in (0, 2, 4, 6):
        w[h], w[h + 1] = _cas(z[h], z[h + 1])
    return w


def _merge_top8(A, B):
    return _bitonic_clean8([jnp.maximum(A[i], B[7 - i]) for i in range(8)])


def _block_top8(v):
    S2 = [_cas(v[2 * j], v[2 * j + 1]) for j in range(8)]
    S4 = [_merge22(S2[2 * j], S2[2 * j + 1]) for j in range(4)]
    S8a = _merge44(S4[0], S4[1])
    S8b = _merge44(S4[2], S4[3])
    return _merge_top8(S8a, S8b)


def _topk_sc(x3):
    B, S, C = x3.shape
    SLAB = 128
    NQ = 4
    SPB = C_SC // SLAB
    SPC = B * SPB // NUM_CORES
    SROWS = S // NQ
    NG = SLAB // LANES
    NCHUNK = SROWS // CHUNK

    x = x3.reshape(B * S, C)
    mesh = plsc.VectorSubcoreMesh(core_axis_name="c", subcore_axis_name="s")

    @functools.partial(
        pl.kernel,
        out_type=jax.ShapeDtypeStruct((B, C_SC * K_TOP), jnp.float32),
        mesh=mesh,
        scratch_types=[
            pltpu.VMEM((CHUNK, SLAB), jnp.float32),
            pltpu.VMEM((CHUNK, SLAB), jnp.float32),
            pltpu.VMEM((K_TOP, SLAB), jnp.float32),
            pltpu.VMEM((K_TOP, SLAB), jnp.float32),
            pltpu.VMEM((SLAB * K_TOP,), jnp.float32),
            pltpu.VMEM_SHARED((NUM_SUBCORES, K_TOP, SLAB), jnp.float32),
            pltpu.SemaphoreType.DMA,
            pltpu.SemaphoreType.DMA,
        ],
        compiler_params=pltpu.CompilerParams(
            use_tc_tiling_on_sc=True, needs_layout_passes=False),
    )
    def k(x_hbm, out_hbm, buf0, buf1, accq, tmpv, accp, shared, sem0, sem1):
        cid = lax.axis_index("c")
        sid = lax.axis_index("s")
        gslab = cid * SPC + sid // NQ
        q = sid % NQ
        b = gslab // SPB
        c0 = (gslab % SPB) * SLAB
        row0 = b * S + q * SROWS

        neg = jnp.full((LANES,), -jnp.inf, dtype=jnp.float32)
        for g in range(NG):
            for kk in range(K_TOP):
                accq[kk, pl.ds(g * LANES, LANES)] = neg

        pltpu.async_copy(
            x_hbm.at[pl.ds(row0, CHUNK), pl.ds(c0, SLAB)], buf0, sem0)
        pltpu.async_copy(
            x_hbm.at[pl.ds(row0 + CHUNK, CHUNK), pl.ds(c0, SLAB)], buf1, sem1)

        def process(buf):
            for g in range(NG):
                def blk_body(i, t, g=g, buf=buf):
                    v = [buf[i * BLOCK + u, pl.ds(g * LANES, LANES)]
                         for u in range(BLOCK)]
                    s = _block_top8(v)
                    return tuple(_merge_top8(list(t), s))

                t = tuple(accq[kk, pl.ds(g * LANES, LANES)]
                          for kk in range(K_TOP))
                t = lax.fori_loop(0, CHUNK // BLOCK, blk_body, t)
                for kk in range(K_TOP):
                    accq[kk, pl.ds(g * LANES, LANES)] = t[kk]

        @pl.loop(0, NCHUNK, step=2)
        def _(ci):
            for j, (buf, sem) in enumerate(((buf0, sem0), (buf1, sem1))):
                cc = ci + j
                pltpu.make_async_copy(
                    x_hbm.at[pl.ds(row0, CHUNK), pl.ds(c0, SLAB)], buf, sem
                ).wait()
                process(buf)

                @pl.when(cc + 2 < NCHUNK)
                def _(buf=buf, sem=sem, cc=cc):
                    pltpu.async_copy(
                        x_hbm.at[pl.ds(row0 + (cc + 2) * CHUNK, CHUNK),
                                 pl.ds(c0, SLAB)],
                        buf, sem)

        pltpu.sync_copy(accq, shared.at[sid])
        plsc.subcore_barrier()

        @pl.when(q == 0)
        def _():
            for j in range(1, NQ):
                pltpu.sync_copy(shared.at[sid + j], tmpv)
                for g in range(NG):
                    a = [accq[kk, pl.ds(g * LANES, LANES)]
                         for kk in range(K_TOP)]
                    bl = [tmpv[kk, pl.ds(g * LANES, LANES)]
                          for kk in range(K_TOP)]
                    m = _merge_top8(a, bl)
                    for kk in range(K_TOP):
                        accq[kk, pl.ds(g * LANES, LANES)] = m[kk]

            iota = lax.iota(jnp.int32, LANES)
            for g in range(NG):
                for kk in range(K_TOP):
                    plsc.store_scatter(
                        accp,
                        [iota * K_TOP + (g * LANES * K_TOP + kk)],
                        accq[kk, pl.ds(g * LANES, LANES)])
            pltpu.sync_copy(
                accp, out_hbm.at[b, pl.ds(c0 * K_TOP, SLAB * K_TOP)])

    return k(x)


def _topk_tc(x3):
    B, S, C = x3.shape
    NCB = (C - C_SC) // 128

    def body(x_ref, o_ref):
        def blk(i, t):
            v = [x_ref[0, pl.ds(i * 8 * BLOCK + 8 * u, 8), :]
                 for u in range(BLOCK)]
            s = _block_top8(v)
            return tuple(_merge_top8(list(t), s))

        t = tuple(jnp.full((8, 128), -jnp.inf, dtype=jnp.float32)
                  for _ in range(K_TOP))
        t = list(lax.fori_loop(0, S // (8 * BLOCK), blk, t))
        for d in (1, 2, 4):
            tsh = [pltpu.roll(u, 8 - d, 0) for u in t]
            t = _merge_top8(t, tsh)
        for kk in range(K_TOP):
            o_ref[0, 0, pl.ds(kk, 1), :] = t[kk][0:1, :]

    out = pl.pallas_call(
        body,
        grid=(B, NCB),
        in_specs=[pl.BlockSpec((1, S, 128),
                               lambda i, j: (i, 0, j + C_SC // 128))],
        out_specs=pl.BlockSpec((1, 1, K_TOP, 128),
                               lambda i, j: (i, j, 0, 0)),
        out_shape=jax.ShapeDtypeStruct((B, NCB, K_TOP, 128), jnp.float32),
    )(x3)
    return jnp.transpose(out, (0, 1, 3, 2)).reshape(B, (C - C_SC) * K_TOP)


def kernel(inputs):
    sc_out = _topk_sc(inputs)
    tc_out = _topk_tc(inputs)
    return jnp.concatenate([sc_out, tc_out], axis=1)
